# baseline (speedup 1.0000x reference)
import functools

import jax
import jax.numpy as jnp
from jax import lax
from jax.experimental import pallas as pl
from jax.experimental.pallas import tpu as pltpu
from jax.experimental.pallas import tpu_sc as plsc

NC = 2
NS = 16
DEG = 32
D = 128
GROW = 128
CHUNK_NODES = 8
CHUNK_ROWS = CHUNK_NODES * DEG
STREAMS_PER_CHUNK = CHUNK_ROWS // GROW
DCH = D // 16
NBUF = 2


def _agg_body(nodes_per_pair, nch_pair, features, idx_all, out, *scratch):
    idx_v = scratch[0]
    bufs = scratch[1:1 + NBUF]
    stage = scratch[1 + NBUF:1 + 2 * NBUF]
    gsems = scratch[1 + 2 * NBUF:1 + 3 * NBUF]
    osems = scratch[1 + 3 * NBUF:1 + 4 * NBUF]

    cid = lax.axis_index("c")
    pair = lax.axis_index("s")
    pltpu.sync_copy(idx_all.at[pair], idx_v)

    my_nch = nch_pair // 2
    chunk0 = cid * my_nch

    def fire_gather(c, b):
        for t in range(STREAMS_PER_CHUNK):
            pltpu.async_copy(
                features.at[idx_v.at[c * STREAMS_PER_CHUNK + t]],
                bufs[b].at[pl.ds(t * GROW, GROW)], gsems[b])

    def wait_gather(c, b):
        for t in range(STREAMS_PER_CHUNK):
            pltpu.make_async_copy(
                features.at[idx_v.at[c * STREAMS_PER_CHUNK + t]],
                bufs[b].at[pl.ds(t * GROW, GROW)], gsems[b]).wait()

    def compute_chunk(b):
        buf = bufs[b]

        def node_body(n, carry):
            row0 = n * DEG
            for dc in range(DCH):
                a = buf[row0, pl.ds(dc * 16, 16)]
                for j in range(1, DEG):
                    a = a + buf[row0 + j, pl.ds(dc * 16, 16)]
                stage[b][n, pl.ds(dc * 16, 16)] = a
            return carry
        lax.fori_loop(0, CHUNK_NODES, node_body, 0)

    def out_slice(c):
        return out.at[pl.ds(pair * nodes_per_pair + c * CHUNK_NODES,
                            CHUNK_NODES)]

    for b in range(NBUF):
        fire_gather(chunk0 + b, b)

    def group_body(g, carry):
        for b in range(NBUF):
            i = g * NBUF + b
            c = chunk0 + i
            wait_gather(c, b)

            @pl.when(g > 0)
            def _():
                pltpu.make_async_copy(stage[b], out_slice(c), osems[b]).wait()

            compute_chunk(b)
            pltpu.async_copy(stage[b], out_slice(c), osems[b])

            @pl.when(i + NBUF < my_nch)
            def _():
                fire_gather(c + NBUF, b)

        return carry

    lax.fori_loop(0, my_nch // NBUF, group_body, 0)
    for b in range(NBUF):
        last_c = chunk0 + my_nch - NBUF + b
        pltpu.make_async_copy(stage[b], out_slice(last_c), osems[b]).wait()


def kernel(features, nodes, to_neighs):
    del nodes
    B = to_neighs.shape[0]
    tn = to_neighs.astype(jnp.int32)
    bp_unit = NS * CHUNK_NODES * 2 * NBUF
    BP = ((B + bp_unit - 1) // bp_unit) * bp_unit
    nodes_per_pair = BP // NS
    nch_pair = nodes_per_pair // CHUNK_NODES
    if BP != B:
        tn = jnp.pad(tn, ((0, BP - B), (0, 0)))
    idx_all = tn.reshape(NS, nodes_per_pair * DEG // GROW, GROW)

    mesh = plsc.VectorSubcoreMesh(core_axis_name="c", subcore_axis_name="s")
    run = pl.kernel(
        functools.partial(_agg_body, nodes_per_pair, nch_pair),
        out_type=jax.ShapeDtypeStruct((BP, D), jnp.float32),
        mesh=mesh,
        scratch_types=(
            [pltpu.VMEM((nodes_per_pair * DEG // GROW, GROW), jnp.int32)]
            + [pltpu.VMEM((CHUNK_ROWS, D), jnp.float32) for _ in range(NBUF)]
            + [pltpu.VMEM((CHUNK_NODES, D), jnp.float32) for _ in range(NBUF)]
            + [pltpu.SemaphoreType.DMA for _ in range(2 * NBUF)]
        ),
    )
    out = run(features, idx_all)
    return out[:B]


# speedup vs baseline: 1.0089x; 1.0089x over previous
import functools

import jax
import jax.numpy as jnp
from jax import lax
from jax.experimental import pallas as pl
from jax.experimental.pallas import tpu as pltpu
from jax.experimental.pallas import tpu_sc as plsc

NC = 2
NS = 16
DEG = 32
D = 128
GROW = 128
CHUNK_NODES = 8
CHUNK_ROWS = CHUNK_NODES * DEG
STREAMS_PER_CHUNK = CHUNK_ROWS // GROW
DCH = D // 16
NBUF = 2

MB_C1 = 0x5A5A0F1E
MB_C2 = 0x33CC55AA
MB_C3 = 0x0F0F5A5A


def _agg_body(nodes_per_pair, nch_pair, features, idx_all, out, mail,
              *scratch):
    idx_v = scratch[0]
    bufs = scratch[1:1 + NBUF]
    stage = scratch[1 + NBUF:1 + 2 * NBUF]
    mail_v = scratch[1 + 2 * NBUF]
    prd_v = scratch[2 + 2 * NBUF]
    gsems = scratch[3 + 2 * NBUF:3 + 3 * NBUF]
    osems = scratch[3 + 3 * NBUF:3 + 4 * NBUF]
    msem_w = scratch[3 + 4 * NBUF]
    msem_r = scratch[4 + 4 * NBUF]

    cid = lax.axis_index("c")
    pair = lax.axis_index("s")
    myrow = cid * NS + pair
    partner_row = (1 - cid) * NS + pair
    pltpu.sync_copy(idx_all.at[pair], idx_v)

    lanes = jax.lax.iota(jnp.int32, 16)
    frame_off = (jnp.where(lanes == 1, MB_C1, 0)
                 + jnp.where(lanes == 2, MB_C2, 0)
                 + jnp.where(lanes == 3, MB_C3, 0))

    def phys(i):
        return jnp.where(cid == 0, i, nch_pair - 1 - i)

    def fire_gather(i, b):
        c = phys(i)
        for t in range(STREAMS_PER_CHUNK):
            pltpu.async_copy(
                features.at[idx_v.at[c * STREAMS_PER_CHUNK + t]],
                bufs[b].at[pl.ds(t * GROW, GROW)], gsems[b])

    def wait_gather(i, b):
        c = phys(i)
        for t in range(STREAMS_PER_CHUNK):
            pltpu.make_async_copy(
                features.at[idx_v.at[c * STREAMS_PER_CHUNK + t]],
                bufs[b].at[pl.ds(t * GROW, GROW)], gsems[b]).wait()

    def compute_chunk(b):
        buf = bufs[b]

        def node_body(n, carry):
            row0 = n * DEG
            for dc in range(DCH):
                a = buf[row0, pl.ds(dc * 16, 16)]
                for j in range(1, DEG):
                    a = a + buf[row0 + j, pl.ds(dc * 16, 16)]
                stage[b][n, pl.ds(dc * 16, 16)] = a
            return carry
        lax.fori_loop(0, CHUNK_NODES, node_body, 0)

    def out_slice(i):
        return out.at[pl.ds(pair * nodes_per_pair + phys(i) * CHUNK_NODES,
                            CHUNK_NODES)]

    for b in range(NBUF):
        fire_gather(b, b)

    def group_work(g):
        for b in range(NBUF):
            i = g * NBUF + b
            wait_gather(i, b)

            @pl.when(g > 0)
            def _():
                pltpu.make_async_copy(stage[b], out_slice(i - NBUF),
                                      osems[b]).wait()

            compute_chunk(b)
            pltpu.async_copy(stage[b], out_slice(i), osems[b])

            @pl.when(i + NBUF < nch_pair)
            def _():
                fire_gather(i + NBUF, b)

        done = (g + 1) * NBUF
        mail_v[pl.ds(0, 16)] = jnp.full((16,), done, jnp.int32) + frame_off
        pltpu.async_copy(mail_v, mail.at[myrow], msem_w)
        pltpu.make_async_copy(mail_v, mail.at[myrow], msem_w).wait()

    def group_body(g, carry):
        processed, partner_seen = carry
        active = g * NBUF + partner_seen < nch_pair

        @pl.when(active)
        def _():
            group_work(g)

        pltpu.sync_copy(mail.at[partner_row], prd_v)
        p16 = prd_v[pl.ds(0, 16)]
        pcount = p16[0]
        ok = ((p16[1] == pcount + MB_C1)
              & (p16[2] == pcount + MB_C2)
              & (p16[3] == pcount + MB_C3))
        pcount = jnp.where(ok, jnp.clip(pcount, 0, nch_pair), 0)
        return (jnp.where(active, g + 1, processed),
                jnp.maximum(partner_seen, pcount))

    processed, _ = lax.fori_loop(0, nch_pair // NBUF, group_body,
                                 (jnp.int32(0), jnp.int32(0)))
    stop = processed * NBUF

    for b in range(NBUF):
        @pl.when(stop + b < nch_pair)
        def _():
            wait_gather(stop + b, b)

        @pl.when(stop - NBUF + b >= 0)
        def _():
            pltpu.make_async_copy(stage[b], out_slice(stop - NBUF + b),
                                  osems[b]).wait()


def kernel(features, nodes, to_neighs):
    del nodes
    B = to_neighs.shape[0]
    tn = to_neighs.astype(jnp.int32)
    bp_unit = NS * CHUNK_NODES * 2 * NBUF
    BP = ((B + bp_unit - 1) // bp_unit) * bp_unit
    nodes_per_pair = BP // NS
    nch_pair = nodes_per_pair // CHUNK_NODES
    if BP != B:
        tn = jnp.pad(tn, ((0, BP - B), (0, 0)))
    idx_all = tn.reshape(NS, nodes_per_pair * DEG // GROW, GROW)

    mesh = plsc.VectorSubcoreMesh(core_axis_name="c", subcore_axis_name="s")
    run = pl.kernel(
        functools.partial(_agg_body, nodes_per_pair, nch_pair),
        out_type=(
            jax.ShapeDtypeStruct((BP, D), jnp.float32),
            jax.ShapeDtypeStruct((NC * NS, D), jnp.int32),
        ),
        mesh=mesh,
        scratch_types=(
            [pltpu.VMEM((nodes_per_pair * DEG // GROW, GROW), jnp.int32)]
            + [pltpu.VMEM((CHUNK_ROWS, D), jnp.float32) for _ in range(NBUF)]
            + [pltpu.VMEM((CHUNK_NODES, D), jnp.float32) for _ in range(NBUF)]
            + [pltpu.VMEM((D,), jnp.int32)]
            + [pltpu.VMEM((D,), jnp.int32)]
            + [pltpu.SemaphoreType.DMA for _ in range(2 * NBUF + 2)]
        ),
    )
    out, _ = run(features, idx_all)
    return out[:B]
